# baseline (speedup 1.0000x reference)
import functools

import jax
import jax.numpy as jnp
from jax import lax
from jax.experimental import pallas as pl
from jax.experimental.pallas import tpu as pltpu
from jax.experimental.pallas import tpu_sc as plsc

_B, _P, _F, _H = 16, 512, 2048, 1024

_BS = 4
_W = 128
_NJ = _F // _W
_TWO_PASS = True


def _idx_kernel(ap_ref, out_ref):
    x = ap_ref[...]
    prev = jnp.concatenate([x[:, :1], x[:, :-1]], axis=1)
    c = (x != prev).astype(jnp.int32)
    k = 1
    while k < _F:
        shifted = jnp.concatenate(
            [jnp.zeros((_B, k), jnp.int32), c[:, : _F - k]], axis=1)
        c = c + shifted
        k *= 2
    row = lax.broadcasted_iota(jnp.int32, (_B, _F), 0)
    out_ref[...] = c + row * _P


def _build_indices(ap):
    return pl.pallas_call(
        _idx_kernel,
        out_shape=jax.ShapeDtypeStruct((_B, _F), jnp.int32),
    )(ap)


_NW = 32
_ROWS_PER_W = (_BS * _F) // _NW
_CHUNK = 64
_N_IT = _ROWS_PER_W // _CHUNK


def _sc_gather(table, gidx):
    mesh = plsc.VectorSubcoreMesh(core_axis_name="c", subcore_axis_name="s")

    @functools.partial(
        pl.kernel,
        mesh=mesh,
        out_type=jax.ShapeDtypeStruct((_BS * _F, _H), jnp.float32),
        scratch_types=[
            pltpu.VMEM((_ROWS_PER_W,), jnp.int32),
            pltpu.VMEM((_CHUNK, _H), jnp.float32),
            pltpu.SemaphoreType.DMA,
        ],
    )
    def k(table_hbm, gidx_hbm, out_hbm, idx_v, rows_v, sem):
        wid = lax.axis_index("s") * 2 + lax.axis_index("c")
        base = wid * _ROWS_PER_W
        pltpu.sync_copy(gidx_hbm.at[pl.ds(base, _ROWS_PER_W)], idx_v)
        for i in range(_N_IT):
            pltpu.async_copy(
                table_hbm.at[idx_v.at[pl.ds(i * _CHUNK, _CHUNK)]],
                rows_v, sem).wait()
            pltpu.sync_copy(rows_v, out_hbm.at[pl.ds(base + i * _CHUNK, _CHUNK)])

    return k(table, gidx)


def _rank1(j, p_blk, bt_blk, wp_ref, wb_ref, wpos_ref, bp_ref, bb_ref, bpos_ref):
    pos = (j * _W + lax.broadcasted_iota(jnp.int32, (_W, 1), 0)
           ).astype(jnp.float32)
    bias = bp_ref[...] + bb_ref[...] + bpos_ref[...]
    return (p_blk * wp_ref[...] + bt_blk * wb_ref[...]
            + pos * wpos_ref[...] + bias)


def _onehot_kernel(gidx_ref, hi_ref, lo_ref, p_ref, bt_ref,
                   wp_ref, wb_ref, wpos_ref, bp_ref, bb_ref, bpos_ref, o_ref):
    bb_i = pl.program_id(0)
    j = pl.program_id(1)
    row0 = (bb_i + _BS) * _P
    gidx = gidx_ref[0]
    iot = row0 + lax.broadcasted_iota(jnp.int32, (_W, _P), 1)
    oh = (gidx == iot).astype(jnp.bfloat16)
    acc = lax.dot_general(oh, hi_ref[0], (((1,), (0,)), ((), ())),
                          preferred_element_type=jnp.float32)
    if _TWO_PASS:
        acc = acc + lax.dot_general(oh, lo_ref[0], (((1,), (0,)), ((), ())),
                                    preferred_element_type=jnp.float32)
    o_ref[0] = acc + _rank1(j, p_ref[0], bt_ref[0], wp_ref, wb_ref,
                            wpos_ref, bp_ref, bb_ref, bpos_ref)


def _onehot_call(gidx3, enc_hi, enc_lo, pitch3, beats3,
                 W_pitch, W_beats, W_pos, b_pitch, b_beats, b_pos):
    vec = lambda: pl.BlockSpec((1, _H), lambda b, j: (0, 0))
    fblk = lambda off: pl.BlockSpec((1, _W, 1), lambda b, j: (b + off, j, 0))
    return pl.pallas_call(
        _onehot_kernel,
        grid=(_B - _BS, _NJ),
        in_specs=[
            pl.BlockSpec((1, _W, 1), lambda b, j: (b, j, 0)),
            pl.BlockSpec((1, _P, _H), lambda b, j: (b + _BS, 0, 0)),
            pl.BlockSpec((1, _P, _H), lambda b, j: (b + _BS, 0, 0)),
            fblk(_BS), fblk(_BS),
            vec(), vec(), vec(), vec(), vec(), vec(),
        ],
        out_specs=pl.BlockSpec((1, _W, _H), lambda b, j: (b + _BS, j, 0)),
        out_shape=jax.ShapeDtypeStruct((_B, _F, _H), jnp.float32),
    )(gidx3, enc_hi, enc_lo, pitch3, beats3,
      W_pitch, W_beats, W_pos, b_pitch, b_beats, b_pos)


def _fuse_kernel(out_in_ref, g_ref, p_ref, bt_ref,
                 wp_ref, wb_ref, wpos_ref, bp_ref, bb_ref, bpos_ref, o_ref):
    j = pl.program_id(1)
    o_ref[0] = g_ref[0] + _rank1(j, p_ref[0], bt_ref[0], wp_ref, wb_ref,
                                 wpos_ref, bp_ref, bb_ref, bpos_ref)


def _fuse_call(out_full, g, pitch3, beats3,
               W_pitch, W_beats, W_pos, b_pitch, b_beats, b_pos):
    vec = lambda: pl.BlockSpec((1, _H), lambda b, j: (0, 0))
    fblk = lambda: pl.BlockSpec((1, _W, 1), lambda b, j: (b, j, 0))
    return pl.pallas_call(
        _fuse_kernel,
        grid=(_BS, _NJ),
        in_specs=[
            pl.BlockSpec((1, _W, _H), lambda b, j: (b, j, 0)),
            pl.BlockSpec((1, _W, _H), lambda b, j: (b, j, 0)),
            fblk(), fblk(),
            vec(), vec(), vec(), vec(), vec(), vec(),
        ],
        out_specs=pl.BlockSpec((1, _W, _H), lambda b, j: (b, j, 0)),
        out_shape=jax.ShapeDtypeStruct((_B, _F, _H), jnp.float32),
        input_output_aliases={0: 0},
    )(out_full, g, pitch3, beats3,
      W_pitch, W_beats, W_pos, b_pitch, b_beats, b_pos)


def kernel(encoder_out, align_phone, pitch, beats,
           W_pitch, b_pitch, W_pos, b_pos, W_beats, b_beats):
    ap = align_phone.astype(jnp.int32)
    gidx = _build_indices(ap)

    table = encoder_out.reshape(_B * _P, _H)
    g = _sc_gather(table, gidx[:_BS].reshape(_BS * _F))

    enc_hi = encoder_out.astype(jnp.bfloat16)
    if _TWO_PASS:
        enc_lo = (encoder_out - enc_hi.astype(jnp.float32)).astype(jnp.bfloat16)
    else:
        enc_lo = enc_hi

    pitch3 = pitch.reshape(_B, _F, 1)
    beats3 = beats.reshape(_B, _F, 1)
    bp = b_pitch.reshape(1, _H)
    bb = b_beats.reshape(1, _H)
    bpos = b_pos.reshape(1, _H)

    out = _onehot_call(gidx[_BS:].reshape(_B - _BS, _F, 1), enc_hi, enc_lo,
                       pitch3, beats3, W_pitch, W_beats, W_pos, bp, bb, bpos)
    out = _fuse_call(out, g.reshape(_BS, _F, _H),
                     pitch3[:_BS], beats3[:_BS],
                     W_pitch, W_beats, W_pos, bp, bb, bpos)
    return out


# speedup vs baseline: 1.7040x; 1.7040x over previous
import functools

import jax
import jax.numpy as jnp
from jax import lax
from jax.experimental import pallas as pl
from jax.experimental.pallas import tpu as pltpu
from jax.experimental.pallas import tpu_sc as plsc

_B, _P, _F, _H = 16, 512, 2048, 1024

_BS = 4
_W = 256
_NJ = _F // _W


def _idx_kernel(ap_ref, out_ref):
    x = ap_ref[...]
    prev = jnp.concatenate([x[:, :1], x[:, :-1]], axis=1)
    c = (x != prev).astype(jnp.int32)
    k = 1
    while k < _F:
        shifted = jnp.concatenate(
            [jnp.zeros((_B, k), jnp.int32), c[:, : _F - k]], axis=1)
        c = c + shifted
        k *= 2
    row = lax.broadcasted_iota(jnp.int32, (_B, _F), 0)
    out_ref[...] = c + row * _P


def _build_indices(ap):
    return pl.pallas_call(
        _idx_kernel,
        out_shape=jax.ShapeDtypeStruct((_B, _F), jnp.int32),
    )(ap)


_NW = 32
_ROWS_PER_W = (_BS * _F) // _NW
_CHUNK = 64
_N_IT = _ROWS_PER_W // _CHUNK


def _sc_gather(table, gidx):
    mesh = plsc.VectorSubcoreMesh(core_axis_name="c", subcore_axis_name="s")

    @functools.partial(
        pl.kernel,
        mesh=mesh,
        out_type=jax.ShapeDtypeStruct((_BS * _F, _H), jnp.float32),
        scratch_types=[
            pltpu.VMEM((_ROWS_PER_W,), jnp.int32),
            pltpu.VMEM((_CHUNK, _H), jnp.float32),
            pltpu.SemaphoreType.DMA,
        ],
    )
    def k(table_hbm, gidx_hbm, out_hbm, idx_v, rows_v, sem):
        wid = lax.axis_index("s") * 2 + lax.axis_index("c")
        base = wid * _ROWS_PER_W
        pltpu.sync_copy(gidx_hbm.at[pl.ds(base, _ROWS_PER_W)], idx_v)
        for i in range(_N_IT):
            pltpu.async_copy(
                table_hbm.at[idx_v.at[pl.ds(i * _CHUNK, _CHUNK)]],
                rows_v, sem).wait()
            pltpu.sync_copy(rows_v, out_hbm.at[pl.ds(base + i * _CHUNK, _CHUNK)])

    return k(table, gidx)


def _rank1(j, p_col, bt_col, wp_ref, wb_ref, wpos_ref, bp_ref, bb_ref, bpos_ref):
    pos = (j * _W + lax.broadcasted_iota(jnp.int32, (_W, 1), 0)
           ).astype(jnp.float32)
    bias = bp_ref[...] + bb_ref[...] + bpos_ref[...]
    return (p_col * wp_ref[...] + bt_col * wb_ref[...]
            + pos * wpos_ref[...] + bias)


def _col(ref, b_abs):
    x = ref[0]
    mask = lax.broadcasted_iota(jnp.int32, (1, _B), 1) == b_abs
    if x.dtype == jnp.int32:
        return jnp.sum(jnp.where(mask, x, 0), axis=1, keepdims=True)
    return jnp.sum(jnp.where(mask, x, 0.0), axis=1, keepdims=True)


def _onehot_kernel(gidx_ref, enc_ref, p_ref, bt_ref,
                   wp_ref, wb_ref, wpos_ref, bp_ref, bb_ref, bpos_ref,
                   o_ref, hi_scr):
    bb_i = pl.program_id(0)
    j = pl.program_id(1)

    @pl.when(j == 0)
    def _():
        hi_scr[...] = enc_ref[0].astype(jnp.bfloat16)

    row0 = (bb_i + _BS) * _P
    gidx = _col(gidx_ref, bb_i + _BS)
    iot = row0 + lax.broadcasted_iota(jnp.int32, (_W, _P), 1)
    oh = (gidx == iot).astype(jnp.bfloat16)
    acc = lax.dot_general(oh, hi_scr[...], (((1,), (0,)), ((), ())),
                          preferred_element_type=jnp.float32)
    o_ref[0] = acc + _rank1(j, _col(p_ref, bb_i + _BS), _col(bt_ref, bb_i + _BS),
                            wp_ref, wb_ref, wpos_ref, bp_ref, bb_ref, bpos_ref)


def _onehot_call(gidxT, enc, pitchT, beatsT,
                 W_pitch, W_beats, W_pos, b_pitch, b_beats, b_pos):
    vec = lambda: pl.BlockSpec((1, _H), lambda b, j: (0, 0))
    colblk = lambda: pl.BlockSpec((1, _W, _B), lambda b, j: (j, 0, 0))
    return pl.pallas_call(
        _onehot_kernel,
        grid=(_B - _BS, _NJ),
        in_specs=[
            colblk(),
            pl.BlockSpec((1, _P, _H), lambda b, j: (b + _BS, 0, 0)),
            colblk(), colblk(),
            vec(), vec(), vec(), vec(), vec(), vec(),
        ],
        out_specs=pl.BlockSpec((1, _W, _H), lambda b, j: (b + _BS, j, 0)),
        out_shape=jax.ShapeDtypeStruct((_B, _F, _H), jnp.float32),
        scratch_shapes=[pltpu.VMEM((_P, _H), jnp.bfloat16)],
    )(gidxT, enc, pitchT, beatsT,
      W_pitch, W_beats, W_pos, b_pitch, b_beats, b_pos)


def _fuse_kernel(out_in_ref, g_ref, p_ref, bt_ref,
                 wp_ref, wb_ref, wpos_ref, bp_ref, bb_ref, bpos_ref, o_ref):
    b = pl.program_id(0)
    j = pl.program_id(1)
    o_ref[0] = g_ref[0] + _rank1(j, _col(p_ref, b), _col(bt_ref, b),
                                 wp_ref, wb_ref, wpos_ref,
                                 bp_ref, bb_ref, bpos_ref)


def _fuse_call(out_full, g, gidxT_unused, pitchT, beatsT,
               W_pitch, W_beats, W_pos, b_pitch, b_beats, b_pos):
    vec = lambda: pl.BlockSpec((1, _H), lambda b, j: (0, 0))
    colblk = lambda: pl.BlockSpec((1, _W, _B), lambda b, j: (j, 0, 0))
    return pl.pallas_call(
        _fuse_kernel,
        grid=(_BS, _NJ),
        in_specs=[
            pl.BlockSpec(memory_space=pl.ANY),
            pl.BlockSpec((1, _W, _H), lambda b, j: (b, j, 0)),
            colblk(), colblk(),
            vec(), vec(), vec(), vec(), vec(), vec(),
        ],
        out_specs=pl.BlockSpec((1, _W, _H), lambda b, j: (b, j, 0)),
        out_shape=jax.ShapeDtypeStruct((_B, _F, _H), jnp.float32),
        input_output_aliases={0: 0},
    )(out_full, g, pitchT, beatsT,
      W_pitch, W_beats, W_pos, b_pitch, b_beats, b_pos)


def kernel(encoder_out, align_phone, pitch, beats,
           W_pitch, b_pitch, W_pos, b_pos, W_beats, b_beats):
    ap = align_phone.astype(jnp.int32)
    gidx = _build_indices(ap)

    table = encoder_out.reshape(_B * _P, _H)
    g = _sc_gather(table, gidx[:_BS].reshape(_BS * _F))

    gidxT = gidx.T.reshape(_NJ, _W, _B)
    pitchT = pitch.T.reshape(_NJ, _W, _B)
    beatsT = beats.T.reshape(_NJ, _W, _B)

    bp = b_pitch.reshape(1, _H)
    bb = b_beats.reshape(1, _H)
    bpos = b_pos.reshape(1, _H)

    out = _onehot_call(gidxT, encoder_out, pitchT, beatsT,
                       W_pitch, W_beats, W_pos, bp, bb, bpos)
    out = _fuse_call(out, g.reshape(_BS, _F, _H), gidxT, pitchT, beatsT,
                     W_pitch, W_beats, W_pos, bp, bb, bpos)
    return out


# speedup vs baseline: 2.0655x; 1.2121x over previous
import functools

import jax
import jax.numpy as jnp
from jax import lax
from jax.experimental import pallas as pl
from jax.experimental.pallas import tpu as pltpu
from jax.experimental.pallas import tpu_sc as plsc

_B, _P, _F, _H = 16, 512, 2048, 1024

_BS = 4
_W = 512
_NJ = _F // _W


def _idx_kernel(ap_ref, out_ref):
    x = ap_ref[...]
    prev = jnp.concatenate([x[:, :1], x[:, :-1]], axis=1)
    c = (x != prev).astype(jnp.int32)
    k = 1
    while k < _F:
        shifted = jnp.concatenate(
            [jnp.zeros((_B, k), jnp.int32), c[:, : _F - k]], axis=1)
        c = c + shifted
        k *= 2
    row = lax.broadcasted_iota(jnp.int32, (_B, _F), 0)
    out_ref[...] = c + row * _P


def _build_indices(ap):
    return pl.pallas_call(
        _idx_kernel,
        out_shape=jax.ShapeDtypeStruct((_B, _F), jnp.int32),
    )(ap)


_NW = 32
_ROWS_PER_W = (_BS * _F) // _NW
_CHUNK = 64
_N_IT = _ROWS_PER_W // _CHUNK


def _sc_gather(table, gidx):
    mesh = plsc.VectorSubcoreMesh(core_axis_name="c", subcore_axis_name="s")

    @functools.partial(
        pl.kernel,
        mesh=mesh,
        out_type=jax.ShapeDtypeStruct((_BS * _F, _H), jnp.float32),
        scratch_types=[
            pltpu.VMEM((_ROWS_PER_W,), jnp.int32),
            pltpu.VMEM((_CHUNK, _H), jnp.float32),
            pltpu.SemaphoreType.DMA,
        ],
    )
    def k(table_hbm, gidx_hbm, out_hbm, idx_v, rows_v, sem):
        wid = lax.axis_index("s") * 2 + lax.axis_index("c")
        base = wid * _ROWS_PER_W
        pltpu.sync_copy(gidx_hbm.at[pl.ds(base, _ROWS_PER_W)], idx_v)
        for i in range(_N_IT):
            pltpu.async_copy(
                table_hbm.at[idx_v.at[pl.ds(i * _CHUNK, _CHUNK)]],
                rows_v, sem).wait()
            pltpu.sync_copy(rows_v, out_hbm.at[pl.ds(base + i * _CHUNK, _CHUNK)])

    return k(table, gidx)


def _rank1(j, p_col, bt_col, wp_ref, wb_ref, wpos_ref, bp_ref, bb_ref, bpos_ref):
    pos = (j * _W + lax.broadcasted_iota(jnp.int32, (_W, 1), 0)
           ).astype(jnp.float32)
    bias = bp_ref[...] + bb_ref[...] + bpos_ref[...]
    return (p_col * wp_ref[...] + bt_col * wb_ref[...]
            + pos * wpos_ref[...] + bias)


def _col(ref, b_abs):
    x = ref[0]
    mask = lax.broadcasted_iota(jnp.int32, (1, _B), 1) == b_abs
    if x.dtype == jnp.int32:
        return jnp.sum(jnp.where(mask, x, 0), axis=1, keepdims=True)
    return jnp.sum(jnp.where(mask, x, 0.0), axis=1, keepdims=True)


def _onehot_kernel(gidx_ref, enc_ref, p_ref, bt_ref,
                   wp_ref, wb_ref, wpos_ref, bp_ref, bb_ref, bpos_ref,
                   o_ref, hi_scr):
    bb_i = pl.program_id(0)
    j = pl.program_id(1)

    @pl.when(j == 0)
    def _():
        hi_scr[...] = enc_ref[0].astype(jnp.bfloat16)

    row0 = (bb_i + _BS) * _P
    gidx = _col(gidx_ref, bb_i + _BS)
    iot = row0 + lax.broadcasted_iota(jnp.int32, (_W, _P), 1)
    oh = (gidx == iot).astype(jnp.bfloat16)
    acc = lax.dot_general(oh, hi_scr[...], (((1,), (0,)), ((), ())),
                          preferred_element_type=jnp.float32)
    o_ref[0] = acc + _rank1(j, _col(p_ref, bb_i + _BS), _col(bt_ref, bb_i + _BS),
                            wp_ref, wb_ref, wpos_ref, bp_ref, bb_ref, bpos_ref)


def _onehot_call(gidxT, enc, pitchT, beatsT,
                 W_pitch, W_beats, W_pos, b_pitch, b_beats, b_pos):
    vec = lambda: pl.BlockSpec((1, _H), lambda b, j: (0, 0))
    colblk = lambda: pl.BlockSpec((1, _W, _B), lambda b, j: (j, 0, 0))
    return pl.pallas_call(
        _onehot_kernel,
        grid=(_B - _BS, _NJ),
        in_specs=[
            colblk(),
            pl.BlockSpec((1, _P, _H), lambda b, j: (b + _BS, 0, 0)),
            colblk(), colblk(),
            vec(), vec(), vec(), vec(), vec(), vec(),
        ],
        out_specs=pl.BlockSpec((1, _W, _H), lambda b, j: (b + _BS, j, 0)),
        out_shape=jax.ShapeDtypeStruct((_B, _F, _H), jnp.float32),
        scratch_shapes=[pltpu.VMEM((_P, _H), jnp.bfloat16)],
    )(gidxT, enc, pitchT, beatsT,
      W_pitch, W_beats, W_pos, b_pitch, b_beats, b_pos)


def _fuse_kernel(out_in_ref, g_ref, p_ref, bt_ref,
                 wp_ref, wb_ref, wpos_ref, bp_ref, bb_ref, bpos_ref, o_ref):
    b = pl.program_id(0)
    j = pl.program_id(1)
    o_ref[0] = g_ref[0] + _rank1(j, _col(p_ref, b), _col(bt_ref, b),
                                 wp_ref, wb_ref, wpos_ref,
                                 bp_ref, bb_ref, bpos_ref)


def _fuse_call(out_full, g, gidxT_unused, pitchT, beatsT,
               W_pitch, W_beats, W_pos, b_pitch, b_beats, b_pos):
    vec = lambda: pl.BlockSpec((1, _H), lambda b, j: (0, 0))
    colblk = lambda: pl.BlockSpec((1, _W, _B), lambda b, j: (j, 0, 0))
    return pl.pallas_call(
        _fuse_kernel,
        grid=(_BS, _NJ),
        in_specs=[
            pl.BlockSpec(memory_space=pl.ANY),
            pl.BlockSpec((1, _W, _H), lambda b, j: (b, j, 0)),
            colblk(), colblk(),
            vec(), vec(), vec(), vec(), vec(), vec(),
        ],
        out_specs=pl.BlockSpec((1, _W, _H), lambda b, j: (b, j, 0)),
        out_shape=jax.ShapeDtypeStruct((_B, _F, _H), jnp.float32),
        input_output_aliases={0: 0},
    )(out_full, g, pitchT, beatsT,
      W_pitch, W_beats, W_pos, b_pitch, b_beats, b_pos)


def kernel(encoder_out, align_phone, pitch, beats,
           W_pitch, b_pitch, W_pos, b_pos, W_beats, b_beats):
    ap = align_phone.astype(jnp.int32)
    gidx = _build_indices(ap)

    table = encoder_out.reshape(_B * _P, _H)
    g = _sc_gather(table, gidx[:_BS].reshape(_BS * _F))

    gidxT = gidx.T.reshape(_NJ, _W, _B)
    pitchT = pitch.T.reshape(_NJ, _W, _B)
    beatsT = beats.T.reshape(_NJ, _W, _B)

    bp = b_pitch.reshape(1, _H)
    bb = b_beats.reshape(1, _H)
    bpos = b_pos.reshape(1, _H)

    out = _onehot_call(gidxT, encoder_out, pitchT, beatsT,
                       W_pitch, W_beats, W_pos, bp, bb, bpos)
    out = _fuse_call(out, g.reshape(_BS, _F, _H), gidxT, pitchT, beatsT,
                     W_pitch, W_beats, W_pos, bp, bb, bpos)
    return out


# speedup vs baseline: 2.2947x; 1.1110x over previous
import functools

import jax
import jax.numpy as jnp
from jax import lax
from jax.experimental import pallas as pl
from jax.experimental.pallas import tpu as pltpu
from jax.experimental.pallas import tpu_sc as plsc

_B, _P, _F, _H = 16, 512, 2048, 1024

_BS = 4
_W = 1024
_NJ = _F // _W


def _idx_kernel(ap_ref, out_ref):
    x = ap_ref[...]
    prev = jnp.concatenate([x[:, :1], x[:, :-1]], axis=1)
    c = (x != prev).astype(jnp.int32)
    k = 1
    while k < _F:
        shifted = jnp.concatenate(
            [jnp.zeros((_B, k), jnp.int32), c[:, : _F - k]], axis=1)
        c = c + shifted
        k *= 2
    row = lax.broadcasted_iota(jnp.int32, (_B, _F), 0)
    out_ref[...] = c + row * _P


def _build_indices(ap):
    return pl.pallas_call(
        _idx_kernel,
        out_shape=jax.ShapeDtypeStruct((_B, _F), jnp.int32),
    )(ap)


_NW = 32
_ROWS_PER_W = (_BS * _F) // _NW
_CHUNK = 64
_N_IT = _ROWS_PER_W // _CHUNK


def _sc_gather(table, gidx):
    mesh = plsc.VectorSubcoreMesh(core_axis_name="c", subcore_axis_name="s")

    @functools.partial(
        pl.kernel,
        mesh=mesh,
        out_type=jax.ShapeDtypeStruct((_BS * _F, _H), jnp.float32),
        scratch_types=[
            pltpu.VMEM((_ROWS_PER_W,), jnp.int32),
            pltpu.VMEM((_CHUNK, _H), jnp.float32),
            pltpu.SemaphoreType.DMA,
        ],
    )
    def k(table_hbm, gidx_hbm, out_hbm, idx_v, rows_v, sem):
        wid = lax.axis_index("s") * 2 + lax.axis_index("c")
        base = wid * _ROWS_PER_W
        pltpu.sync_copy(gidx_hbm.at[pl.ds(base, _ROWS_PER_W)], idx_v)
        for i in range(_N_IT):
            pltpu.async_copy(
                table_hbm.at[idx_v.at[pl.ds(i * _CHUNK, _CHUNK)]],
                rows_v, sem).wait()
            pltpu.sync_copy(rows_v, out_hbm.at[pl.ds(base + i * _CHUNK, _CHUNK)])

    return k(table, gidx)


def _rank1(j, p_col, bt_col, wp_ref, wb_ref, wpos_ref, bp_ref, bb_ref, bpos_ref):
    pos = (j * _W + lax.broadcasted_iota(jnp.int32, (_W, 1), 0)
           ).astype(jnp.float32)
    bias = bp_ref[...] + bb_ref[...] + bpos_ref[...]
    return (p_col * wp_ref[...] + bt_col * wb_ref[...]
            + pos * wpos_ref[...] + bias)


def _col(ref, b_abs):
    x = ref[0]
    mask = lax.broadcasted_iota(jnp.int32, (1, _B), 1) == b_abs
    if x.dtype == jnp.int32:
        return jnp.sum(jnp.where(mask, x, 0), axis=1, keepdims=True)
    return jnp.sum(jnp.where(mask, x, 0.0), axis=1, keepdims=True)


def _onehot_kernel(gidx_ref, enc_ref, p_ref, bt_ref,
                   wp_ref, wb_ref, wpos_ref, bp_ref, bb_ref, bpos_ref,
                   o_ref, hi_scr):
    bb_i = pl.program_id(0)
    j = pl.program_id(1)

    @pl.when(j == 0)
    def _():
        hi_scr[...] = enc_ref[0].astype(jnp.bfloat16)

    row0 = (bb_i + _BS) * _P
    gidx = _col(gidx_ref, bb_i + _BS)
    iot = row0 + lax.broadcasted_iota(jnp.int32, (_W, _P), 1)
    oh = (gidx == iot).astype(jnp.bfloat16)
    acc = lax.dot_general(oh, hi_scr[...], (((1,), (0,)), ((), ())),
                          preferred_element_type=jnp.float32)
    o_ref[0] = acc + _rank1(j, _col(p_ref, bb_i + _BS), _col(bt_ref, bb_i + _BS),
                            wp_ref, wb_ref, wpos_ref, bp_ref, bb_ref, bpos_ref)


def _onehot_call(gidxT, enc, pitchT, beatsT,
                 W_pitch, W_beats, W_pos, b_pitch, b_beats, b_pos):
    vec = lambda: pl.BlockSpec((1, _H), lambda b, j: (0, 0))
    colblk = lambda: pl.BlockSpec((1, _W, _B), lambda b, j: (j, 0, 0))
    return pl.pallas_call(
        _onehot_kernel,
        grid=(_B - _BS, _NJ),
        in_specs=[
            colblk(),
            pl.BlockSpec((1, _P, _H), lambda b, j: (b + _BS, 0, 0)),
            colblk(), colblk(),
            vec(), vec(), vec(), vec(), vec(), vec(),
        ],
        out_specs=pl.BlockSpec((1, _W, _H), lambda b, j: (b + _BS, j, 0)),
        out_shape=jax.ShapeDtypeStruct((_B, _F, _H), jnp.float32),
        scratch_shapes=[pltpu.VMEM((_P, _H), jnp.bfloat16)],
    )(gidxT, enc, pitchT, beatsT,
      W_pitch, W_beats, W_pos, b_pitch, b_beats, b_pos)


def _fuse_kernel(out_in_ref, g_ref, p_ref, bt_ref,
                 wp_ref, wb_ref, wpos_ref, bp_ref, bb_ref, bpos_ref, o_ref):
    b = pl.program_id(0)
    j = pl.program_id(1)
    o_ref[0] = g_ref[0] + _rank1(j, _col(p_ref, b), _col(bt_ref, b),
                                 wp_ref, wb_ref, wpos_ref,
                                 bp_ref, bb_ref, bpos_ref)


def _fuse_call(out_full, g, gidxT_unused, pitchT, beatsT,
               W_pitch, W_beats, W_pos, b_pitch, b_beats, b_pos):
    vec = lambda: pl.BlockSpec((1, _H), lambda b, j: (0, 0))
    colblk = lambda: pl.BlockSpec((1, _W, _B), lambda b, j: (j, 0, 0))
    return pl.pallas_call(
        _fuse_kernel,
        grid=(_BS, _NJ),
        in_specs=[
            pl.BlockSpec(memory_space=pl.ANY),
            pl.BlockSpec((1, _W, _H), lambda b, j: (b, j, 0)),
            colblk(), colblk(),
            vec(), vec(), vec(), vec(), vec(), vec(),
        ],
        out_specs=pl.BlockSpec((1, _W, _H), lambda b, j: (b, j, 0)),
        out_shape=jax.ShapeDtypeStruct((_B, _F, _H), jnp.float32),
        input_output_aliases={0: 0},
    )(out_full, g, pitchT, beatsT,
      W_pitch, W_beats, W_pos, b_pitch, b_beats, b_pos)


def kernel(encoder_out, align_phone, pitch, beats,
           W_pitch, b_pitch, W_pos, b_pos, W_beats, b_beats):
    ap = align_phone.astype(jnp.int32)
    gidx = _build_indices(ap)

    table = encoder_out.reshape(_B * _P, _H)
    g = _sc_gather(table, gidx[:_BS].reshape(_BS * _F))

    gidxT = gidx.T.reshape(_NJ, _W, _B)
    pitchT = pitch.T.reshape(_NJ, _W, _B)
    beatsT = beats.T.reshape(_NJ, _W, _B)

    bp = b_pitch.reshape(1, _H)
    bb = b_beats.reshape(1, _H)
    bpos = b_pos.reshape(1, _H)

    out = _onehot_call(gidxT, encoder_out, pitchT, beatsT,
                       W_pitch, W_beats, W_pos, bp, bb, bpos)
    out = _fuse_call(out, g.reshape(_BS, _F, _H), gidxT, pitchT, beatsT,
                     W_pitch, W_beats, W_pos, bp, bb, bpos)
    return out


# speedup vs baseline: 2.7063x; 1.1794x over previous
import functools

import jax
import jax.numpy as jnp
from jax import lax
from jax.experimental import pallas as pl
from jax.experimental.pallas import tpu as pltpu
from jax.experimental.pallas import tpu_sc as plsc

_B, _P, _F, _H = 16, 512, 2048, 1024

_BS = 4
_W = 2048
_NJ = _F // _W


def _idx_kernel(ap_ref, out_ref):
    x = ap_ref[...]
    prev = jnp.concatenate([x[:, :1], x[:, :-1]], axis=1)
    c = (x != prev).astype(jnp.int32)
    k = 1
    while k < _F:
        shifted = jnp.concatenate(
            [jnp.zeros((_B, k), jnp.int32), c[:, : _F - k]], axis=1)
        c = c + shifted
        k *= 2
    row = lax.broadcasted_iota(jnp.int32, (_B, _F), 0)
    out_ref[...] = c + row * _P


def _build_indices(ap):
    return pl.pallas_call(
        _idx_kernel,
        out_shape=jax.ShapeDtypeStruct((_B, _F), jnp.int32),
    )(ap)


_NW = 32
_ROWS_PER_W = (_BS * _F) // _NW
_CHUNK = 64
_N_IT = _ROWS_PER_W // _CHUNK


def _sc_gather(table, gidx):
    mesh = plsc.VectorSubcoreMesh(core_axis_name="c", subcore_axis_name="s")

    @functools.partial(
        pl.kernel,
        mesh=mesh,
        out_type=jax.ShapeDtypeStruct((_BS * _F, _H), jnp.float32),
        scratch_types=[
            pltpu.VMEM((_ROWS_PER_W,), jnp.int32),
            pltpu.VMEM((_CHUNK, _H), jnp.float32),
            pltpu.SemaphoreType.DMA,
        ],
    )
    def k(table_hbm, gidx_hbm, out_hbm, idx_v, rows_v, sem):
        wid = lax.axis_index("s") * 2 + lax.axis_index("c")
        base = wid * _ROWS_PER_W
        pltpu.sync_copy(gidx_hbm.at[pl.ds(base, _ROWS_PER_W)], idx_v)
        for i in range(_N_IT):
            pltpu.async_copy(
                table_hbm.at[idx_v.at[pl.ds(i * _CHUNK, _CHUNK)]],
                rows_v, sem).wait()
            pltpu.sync_copy(rows_v, out_hbm.at[pl.ds(base + i * _CHUNK, _CHUNK)])

    return k(table, gidx)


def _rank1(j, p_col, bt_col, wp_ref, wb_ref, wpos_ref, bp_ref, bb_ref, bpos_ref):
    pos = (j * _W + lax.broadcasted_iota(jnp.int32, (_W, 1), 0)
           ).astype(jnp.float32)
    bias = bp_ref[...] + bb_ref[...] + bpos_ref[...]
    return (p_col * wp_ref[...] + bt_col * wb_ref[...]
            + pos * wpos_ref[...] + bias)


def _col(ref, b_abs):
    x = ref[0]
    mask = lax.broadcasted_iota(jnp.int32, (1, _B), 1) == b_abs
    if x.dtype == jnp.int32:
        return jnp.sum(jnp.where(mask, x, 0), axis=1, keepdims=True)
    return jnp.sum(jnp.where(mask, x, 0.0), axis=1, keepdims=True)


def _onehot_kernel(gidx_ref, enc_ref, p_ref, bt_ref,
                   wp_ref, wb_ref, wpos_ref, bp_ref, bb_ref, bpos_ref,
                   o_ref, hi_scr):
    bb_i = pl.program_id(0)
    j = pl.program_id(1)

    @pl.when(j == 0)
    def _():
        hi_scr[...] = enc_ref[0].astype(jnp.bfloat16)

    row0 = (bb_i + _BS) * _P
    gidx = _col(gidx_ref, bb_i + _BS)
    iot = row0 + lax.broadcasted_iota(jnp.int32, (_W, _P), 1)
    oh = (gidx == iot).astype(jnp.bfloat16)
    acc = lax.dot_general(oh, hi_scr[...], (((1,), (0,)), ((), ())),
                          preferred_element_type=jnp.float32)
    o_ref[0] = acc + _rank1(j, _col(p_ref, bb_i + _BS), _col(bt_ref, bb_i + _BS),
                            wp_ref, wb_ref, wpos_ref, bp_ref, bb_ref, bpos_ref)


def _onehot_call(gidxT, enc, pitchT, beatsT,
                 W_pitch, W_beats, W_pos, b_pitch, b_beats, b_pos):
    vec = lambda: pl.BlockSpec((1, _H), lambda b, j: (0, 0))
    colblk = lambda: pl.BlockSpec((1, _W, _B), lambda b, j: (j, 0, 0))
    return pl.pallas_call(
        _onehot_kernel,
        grid=(_B - _BS, _NJ),
        in_specs=[
            colblk(),
            pl.BlockSpec((1, _P, _H), lambda b, j: (b + _BS, 0, 0)),
            colblk(), colblk(),
            vec(), vec(), vec(), vec(), vec(), vec(),
        ],
        out_specs=pl.BlockSpec((1, _W, _H), lambda b, j: (b + _BS, j, 0)),
        out_shape=jax.ShapeDtypeStruct((_B, _F, _H), jnp.float32),
        scratch_shapes=[pltpu.VMEM((_P, _H), jnp.bfloat16)],
    )(gidxT, enc, pitchT, beatsT,
      W_pitch, W_beats, W_pos, b_pitch, b_beats, b_pos)


def _fuse_kernel(out_in_ref, g_ref, p_ref, bt_ref,
                 wp_ref, wb_ref, wpos_ref, bp_ref, bb_ref, bpos_ref, o_ref):
    b = pl.program_id(0)
    j = pl.program_id(1)
    o_ref[0] = g_ref[0] + _rank1(j, _col(p_ref, b), _col(bt_ref, b),
                                 wp_ref, wb_ref, wpos_ref,
                                 bp_ref, bb_ref, bpos_ref)


def _fuse_call(out_full, g, gidxT_unused, pitchT, beatsT,
               W_pitch, W_beats, W_pos, b_pitch, b_beats, b_pos):
    vec = lambda: pl.BlockSpec((1, _H), lambda b, j: (0, 0))
    colblk = lambda: pl.BlockSpec((1, _W, _B), lambda b, j: (j, 0, 0))
    return pl.pallas_call(
        _fuse_kernel,
        grid=(_BS, _NJ),
        in_specs=[
            pl.BlockSpec(memory_space=pl.ANY),
            pl.BlockSpec((1, _W, _H), lambda b, j: (b, j, 0)),
            colblk(), colblk(),
            vec(), vec(), vec(), vec(), vec(), vec(),
        ],
        out_specs=pl.BlockSpec((1, _W, _H), lambda b, j: (b, j, 0)),
        out_shape=jax.ShapeDtypeStruct((_B, _F, _H), jnp.float32),
        input_output_aliases={0: 0},
    )(out_full, g, pitchT, beatsT,
      W_pitch, W_beats, W_pos, b_pitch, b_beats, b_pos)


def kernel(encoder_out, align_phone, pitch, beats,
           W_pitch, b_pitch, W_pos, b_pos, W_beats, b_beats):
    ap = align_phone.astype(jnp.int32)
    gidx = _build_indices(ap)

    table = encoder_out.reshape(_B * _P, _H)
    g = _sc_gather(table, gidx[:_BS].reshape(_BS * _F))

    gidxT = gidx.T.reshape(_NJ, _W, _B)
    pitchT = pitch.T.reshape(_NJ, _W, _B)
    beatsT = beats.T.reshape(_NJ, _W, _B)

    bp = b_pitch.reshape(1, _H)
    bb = b_beats.reshape(1, _H)
    bpos = b_pos.reshape(1, _H)

    out = _onehot_call(gidxT, encoder_out, pitchT, beatsT,
                       W_pitch, W_beats, W_pos, bp, bb, bpos)
    out = _fuse_call(out, g.reshape(_BS, _F, _H), gidxT, pitchT, beatsT,
                     W_pitch, W_beats, W_pos, bp, bb, bpos)
    return out


# speedup vs baseline: 3.1780x; 1.1743x over previous
import functools

import jax
import jax.numpy as jnp
from jax import lax
from jax.experimental import pallas as pl
from jax.experimental.pallas import tpu as pltpu
from jax.experimental.pallas import tpu_sc as plsc

_B, _P, _F, _H = 16, 512, 2048, 1024

_BS = 2
_W = 2048
_NJ = _F // _W


def _idx_kernel(ap_ref, out_ref):
    x = ap_ref[...]
    prev = jnp.concatenate([x[:, :1], x[:, :-1]], axis=1)
    c = (x != prev).astype(jnp.int32)
    k = 1
    while k < _F:
        shifted = jnp.concatenate(
            [jnp.zeros((_B, k), jnp.int32), c[:, : _F - k]], axis=1)
        c = c + shifted
        k *= 2
    row = lax.broadcasted_iota(jnp.int32, (_B, _F), 0)
    out_ref[...] = c + row * _P


def _build_indices(ap):
    return pl.pallas_call(
        _idx_kernel,
        out_shape=jax.ShapeDtypeStruct((_B, _F), jnp.int32),
    )(ap)


_NW = 32
_ROWS_PER_W = (_BS * _F) // _NW
_CHUNK = 64
_N_IT = _ROWS_PER_W // _CHUNK


def _sc_gather(table, gidx):
    mesh = plsc.VectorSubcoreMesh(core_axis_name="c", subcore_axis_name="s")

    @functools.partial(
        pl.kernel,
        mesh=mesh,
        out_type=jax.ShapeDtypeStruct((_BS * _F, _H), jnp.float32),
        scratch_types=[
            pltpu.VMEM((_ROWS_PER_W,), jnp.int32),
            pltpu.VMEM((_CHUNK, _H), jnp.float32),
            pltpu.SemaphoreType.DMA,
        ],
    )
    def k(table_hbm, gidx_hbm, out_hbm, idx_v, rows_v, sem):
        wid = lax.axis_index("s") * 2 + lax.axis_index("c")
        base = wid * _ROWS_PER_W
        pltpu.sync_copy(gidx_hbm.at[pl.ds(base, _ROWS_PER_W)], idx_v)
        for i in range(_N_IT):
            pltpu.async_copy(
                table_hbm.at[idx_v.at[pl.ds(i * _CHUNK, _CHUNK)]],
                rows_v, sem).wait()
            pltpu.sync_copy(rows_v, out_hbm.at[pl.ds(base + i * _CHUNK, _CHUNK)])

    return k(table, gidx)


def _rank1(j, p_col, bt_col, wp_ref, wb_ref, wpos_ref, bp_ref, bb_ref, bpos_ref):
    pos = (j * _W + lax.broadcasted_iota(jnp.int32, (_W, 1), 0)
           ).astype(jnp.float32)
    bias = bp_ref[...] + bb_ref[...] + bpos_ref[...]
    return (p_col * wp_ref[...] + bt_col * wb_ref[...]
            + pos * wpos_ref[...] + bias)


def _col(ref, b_abs):
    x = ref[0]
    mask = lax.broadcasted_iota(jnp.int32, (1, _B), 1) == b_abs
    if x.dtype == jnp.int32:
        return jnp.sum(jnp.where(mask, x, 0), axis=1, keepdims=True)
    return jnp.sum(jnp.where(mask, x, 0.0), axis=1, keepdims=True)


def _onehot_kernel(gidx_ref, enc_ref, p_ref, bt_ref,
                   wp_ref, wb_ref, wpos_ref, bp_ref, bb_ref, bpos_ref,
                   o_ref, hi_scr):
    bb_i = pl.program_id(0)
    j = pl.program_id(1)

    @pl.when(j == 0)
    def _():
        hi_scr[...] = enc_ref[0].astype(jnp.bfloat16)

    row0 = (bb_i + _BS) * _P
    gidx = _col(gidx_ref, bb_i + _BS)
    iot = row0 + lax.broadcasted_iota(jnp.int32, (_W, _P), 1)
    oh = (gidx == iot).astype(jnp.bfloat16)
    acc = lax.dot_general(oh, hi_scr[...], (((1,), (0,)), ((), ())),
                          preferred_element_type=jnp.float32)
    o_ref[0] = acc + _rank1(j, _col(p_ref, bb_i + _BS), _col(bt_ref, bb_i + _BS),
                            wp_ref, wb_ref, wpos_ref, bp_ref, bb_ref, bpos_ref)


def _onehot_call(gidxT, enc, pitchT, beatsT,
                 W_pitch, W_beats, W_pos, b_pitch, b_beats, b_pos):
    vec = lambda: pl.BlockSpec((1, _H), lambda b, j: (0, 0))
    colblk = lambda: pl.BlockSpec((1, _W, _B), lambda b, j: (j, 0, 0))
    return pl.pallas_call(
        _onehot_kernel,
        grid=(_B - _BS, _NJ),
        in_specs=[
            colblk(),
            pl.BlockSpec((1, _P, _H), lambda b, j: (b + _BS, 0, 0)),
            colblk(), colblk(),
            vec(), vec(), vec(), vec(), vec(), vec(),
        ],
        out_specs=pl.BlockSpec((1, _W, _H), lambda b, j: (b + _BS, j, 0)),
        out_shape=jax.ShapeDtypeStruct((_B, _F, _H), jnp.float32),
        scratch_shapes=[pltpu.VMEM((_P, _H), jnp.bfloat16)],
    )(gidxT, enc, pitchT, beatsT,
      W_pitch, W_beats, W_pos, b_pitch, b_beats, b_pos)


def _fuse_kernel(out_in_ref, g_ref, p_ref, bt_ref,
                 wp_ref, wb_ref, wpos_ref, bp_ref, bb_ref, bpos_ref, o_ref):
    b = pl.program_id(0)
    j = pl.program_id(1)
    o_ref[0] = g_ref[0] + _rank1(j, _col(p_ref, b), _col(bt_ref, b),
                                 wp_ref, wb_ref, wpos_ref,
                                 bp_ref, bb_ref, bpos_ref)


def _fuse_call(out_full, g, gidxT_unused, pitchT, beatsT,
               W_pitch, W_beats, W_pos, b_pitch, b_beats, b_pos):
    vec = lambda: pl.BlockSpec((1, _H), lambda b, j: (0, 0))
    colblk = lambda: pl.BlockSpec((1, _W, _B), lambda b, j: (j, 0, 0))
    return pl.pallas_call(
        _fuse_kernel,
        grid=(_BS, _NJ),
        in_specs=[
            pl.BlockSpec(memory_space=pl.ANY),
            pl.BlockSpec((1, _W, _H), lambda b, j: (b, j, 0)),
            colblk(), colblk(),
            vec(), vec(), vec(), vec(), vec(), vec(),
        ],
        out_specs=pl.BlockSpec((1, _W, _H), lambda b, j: (b, j, 0)),
        out_shape=jax.ShapeDtypeStruct((_B, _F, _H), jnp.float32),
        input_output_aliases={0: 0},
    )(out_full, g, pitchT, beatsT,
      W_pitch, W_beats, W_pos, b_pitch, b_beats, b_pos)


def kernel(encoder_out, align_phone, pitch, beats,
           W_pitch, b_pitch, W_pos, b_pos, W_beats, b_beats):
    ap = align_phone.astype(jnp.int32)
    gidx = _build_indices(ap)

    table = encoder_out.reshape(_B * _P, _H)
    g = _sc_gather(table, gidx[:_BS].reshape(_BS * _F))

    gidxT = gidx.T.reshape(_NJ, _W, _B)
    pitchT = pitch.T.reshape(_NJ, _W, _B)
    beatsT = beats.T.reshape(_NJ, _W, _B)

    bp = b_pitch.reshape(1, _H)
    bb = b_beats.reshape(1, _H)
    bpos = b_pos.reshape(1, _H)

    out = _onehot_call(gidxT, encoder_out, pitchT, beatsT,
                       W_pitch, W_beats, W_pos, bp, bb, bpos)
    out = _fuse_call(out, g.reshape(_BS, _F, _H), gidxT, pitchT, beatsT,
                     W_pitch, W_beats, W_pos, bp, bb, bpos)
    return out


# speedup vs baseline: 3.2561x; 1.0246x over previous
import functools

import jax
import jax.numpy as jnp
from jax import lax
from jax.experimental import pallas as pl
from jax.experimental.pallas import tpu as pltpu
from jax.experimental.pallas import tpu_sc as plsc

_B, _P, _F, _H = 16, 512, 2048, 1024

_BS = 1
_W = 2048
_NJ = _F // _W


def _idx_kernel(ap_ref, out_ref):
    x = ap_ref[...]
    prev = jnp.concatenate([x[:, :1], x[:, :-1]], axis=1)
    c = (x != prev).astype(jnp.int32)
    k = 1
    while k < _F:
        shifted = jnp.concatenate(
            [jnp.zeros((_B, k), jnp.int32), c[:, : _F - k]], axis=1)
        c = c + shifted
        k *= 2
    row = lax.broadcasted_iota(jnp.int32, (_B, _F), 0)
    out_ref[...] = c + row * _P


def _build_indices(ap):
    return pl.pallas_call(
        _idx_kernel,
        out_shape=jax.ShapeDtypeStruct((_B, _F), jnp.int32),
    )(ap)


_NW = 32
_ROWS_PER_W = (_BS * _F) // _NW
_CHUNK = 64
_N_IT = _ROWS_PER_W // _CHUNK


def _sc_gather(table, gidx):
    mesh = plsc.VectorSubcoreMesh(core_axis_name="c", subcore_axis_name="s")

    @functools.partial(
        pl.kernel,
        mesh=mesh,
        out_type=jax.ShapeDtypeStruct((_BS * _F, _H), jnp.float32),
        scratch_types=[
            pltpu.VMEM((_ROWS_PER_W,), jnp.int32),
            pltpu.VMEM((_CHUNK, _H), jnp.float32),
            pltpu.SemaphoreType.DMA,
        ],
    )
    def k(table_hbm, gidx_hbm, out_hbm, idx_v, rows_v, sem):
        wid = lax.axis_index("s") * 2 + lax.axis_index("c")
        base = wid * _ROWS_PER_W
        pltpu.sync_copy(gidx_hbm.at[pl.ds(base, _ROWS_PER_W)], idx_v)
        for i in range(_N_IT):
            pltpu.async_copy(
                table_hbm.at[idx_v.at[pl.ds(i * _CHUNK, _CHUNK)]],
                rows_v, sem).wait()
            pltpu.sync_copy(rows_v, out_hbm.at[pl.ds(base + i * _CHUNK, _CHUNK)])

    return k(table, gidx)


def _rank1(j, p_col, bt_col, wp_ref, wb_ref, wpos_ref, bp_ref, bb_ref, bpos_ref):
    pos = (j * _W + lax.broadcasted_iota(jnp.int32, (_W, 1), 0)
           ).astype(jnp.float32)
    bias = bp_ref[...] + bb_ref[...] + bpos_ref[...]
    return (p_col * wp_ref[...] + bt_col * wb_ref[...]
            + pos * wpos_ref[...] + bias)


def _col(ref, b_abs):
    x = ref[0]
    mask = lax.broadcasted_iota(jnp.int32, (1, _B), 1) == b_abs
    if x.dtype == jnp.int32:
        return jnp.sum(jnp.where(mask, x, 0), axis=1, keepdims=True)
    return jnp.sum(jnp.where(mask, x, 0.0), axis=1, keepdims=True)


def _onehot_kernel(gidx_ref, enc_ref, p_ref, bt_ref,
                   wp_ref, wb_ref, wpos_ref, bp_ref, bb_ref, bpos_ref,
                   o_ref, hi_scr):
    bb_i = pl.program_id(0)
    j = pl.program_id(1)

    @pl.when(j == 0)
    def _():
        hi_scr[...] = enc_ref[0].astype(jnp.bfloat16)

    row0 = (bb_i + _BS) * _P
    gidx = _col(gidx_ref, bb_i + _BS)
    iot = row0 + lax.broadcasted_iota(jnp.int32, (_W, _P), 1)
    oh = (gidx == iot).astype(jnp.bfloat16)
    acc = lax.dot_general(oh, hi_scr[...], (((1,), (0,)), ((), ())),
                          preferred_element_type=jnp.float32)
    o_ref[0] = acc + _rank1(j, _col(p_ref, bb_i + _BS), _col(bt_ref, bb_i + _BS),
                            wp_ref, wb_ref, wpos_ref, bp_ref, bb_ref, bpos_ref)


def _onehot_call(gidxT, enc, pitchT, beatsT,
                 W_pitch, W_beats, W_pos, b_pitch, b_beats, b_pos):
    vec = lambda: pl.BlockSpec((1, _H), lambda b, j: (0, 0))
    colblk = lambda: pl.BlockSpec((1, _W, _B), lambda b, j: (j, 0, 0))
    return pl.pallas_call(
        _onehot_kernel,
        grid=(_B - _BS, _NJ),
        in_specs=[
            colblk(),
            pl.BlockSpec((1, _P, _H), lambda b, j: (b + _BS, 0, 0)),
            colblk(), colblk(),
            vec(), vec(), vec(), vec(), vec(), vec(),
        ],
        out_specs=pl.BlockSpec((1, _W, _H), lambda b, j: (b + _BS, j, 0)),
        out_shape=jax.ShapeDtypeStruct((_B, _F, _H), jnp.float32),
        scratch_shapes=[pltpu.VMEM((_P, _H), jnp.bfloat16)],
    )(gidxT, enc, pitchT, beatsT,
      W_pitch, W_beats, W_pos, b_pitch, b_beats, b_pos)


def _fuse_kernel(out_in_ref, g_ref, p_ref, bt_ref,
                 wp_ref, wb_ref, wpos_ref, bp_ref, bb_ref, bpos_ref, o_ref):
    b = pl.program_id(0)
    j = pl.program_id(1)
    o_ref[0] = g_ref[0] + _rank1(j, _col(p_ref, b), _col(bt_ref, b),
                                 wp_ref, wb_ref, wpos_ref,
                                 bp_ref, bb_ref, bpos_ref)


def _fuse_call(out_full, g, gidxT_unused, pitchT, beatsT,
               W_pitch, W_beats, W_pos, b_pitch, b_beats, b_pos):
    vec = lambda: pl.BlockSpec((1, _H), lambda b, j: (0, 0))
    colblk = lambda: pl.BlockSpec((1, _W, _B), lambda b, j: (j, 0, 0))
    return pl.pallas_call(
        _fuse_kernel,
        grid=(_BS, _NJ),
        in_specs=[
            pl.BlockSpec(memory_space=pl.ANY),
            pl.BlockSpec((1, _W, _H), lambda b, j: (b, j, 0)),
            colblk(), colblk(),
            vec(), vec(), vec(), vec(), vec(), vec(),
        ],
        out_specs=pl.BlockSpec((1, _W, _H), lambda b, j: (b, j, 0)),
        out_shape=jax.ShapeDtypeStruct((_B, _F, _H), jnp.float32),
        input_output_aliases={0: 0},
    )(out_full, g, pitchT, beatsT,
      W_pitch, W_beats, W_pos, b_pitch, b_beats, b_pos)


def kernel(encoder_out, align_phone, pitch, beats,
           W_pitch, b_pitch, W_pos, b_pos, W_beats, b_beats):
    ap = align_phone.astype(jnp.int32)
    gidx = _build_indices(ap)

    table = encoder_out.reshape(_B * _P, _H)
    g = _sc_gather(table, gidx[:_BS].reshape(_BS * _F))

    gidxT = gidx.T.reshape(_NJ, _W, _B)
    pitchT = pitch.T.reshape(_NJ, _W, _B)
    beatsT = beats.T.reshape(_NJ, _W, _B)

    bp = b_pitch.reshape(1, _H)
    bb = b_beats.reshape(1, _H)
    bpos = b_pos.reshape(1, _H)

    out = _onehot_call(gidxT, encoder_out, pitchT, beatsT,
                       W_pitch, W_beats, W_pos, bp, bb, bpos)
    out = _fuse_call(out, g.reshape(_BS, _F, _H), gidxT, pitchT, beatsT,
                     W_pitch, W_beats, W_pos, bp, bb, bpos)
    return out
